# Initial kernel scaffold; baseline (speedup 1.0000x reference)
import jax
import jax.numpy as jnp
from jax.experimental import pallas as pl


def kernel(x, Wg, bg):
    raise NotImplementedError("write your pallas kernel here")


# speedup vs baseline: 1.1505x; 1.1505x over previous
import jax
import jax.numpy as jnp
from jax.experimental import pallas as pl

N_TOKENS = 16384
D_MODEL = 1024
BLK = 512


def _body(x_ref, wg_ref, bg_ref, o0_ref, o1_ref, o2_ref):
    xb = x_ref[...]
    z = jnp.dot(xb, wg_ref[...], preferred_element_type=jnp.float32) + bg_ref[...]
    m0 = (z[:, 0:1] > 0.0).astype(jnp.float32)
    m1 = (z[:, 1:2] > 0.0).astype(jnp.float32)
    a = xb * m0
    b = xb * m1
    o0_ref[...] = a
    o1_ref[...] = b
    o2_ref[...] = a + b


def kernel(x, Wg, bg):
    n, d = x.shape
    wg_pad = jnp.zeros((d, 128), jnp.float32).at[:, : Wg.shape[1]].set(Wg)
    bg_pad = jnp.zeros((1, 128), jnp.float32).at[0, : bg.shape[0]].set(bg)
    out_shape = jax.ShapeDtypeStruct((n, d), jnp.float32)
    grid = (n // BLK,)
    o0, o1, o2 = pl.pallas_call(
        _body,
        grid=grid,
        in_specs=[
            pl.BlockSpec((BLK, d), lambda i: (i, 0)),
            pl.BlockSpec((d, 128), lambda i: (0, 0)),
            pl.BlockSpec((1, 128), lambda i: (0, 0)),
        ],
        out_specs=[
            pl.BlockSpec((BLK, d), lambda i: (i, 0)),
            pl.BlockSpec((BLK, d), lambda i: (i, 0)),
            pl.BlockSpec((BLK, d), lambda i: (i, 0)),
        ],
        out_shape=[out_shape, out_shape, out_shape],
    )(x, wg_pad, bg_pad)
    return (o0, o1, o2)


# speedup vs baseline: 1.2000x; 1.0430x over previous
import jax
import jax.numpy as jnp
from jax.experimental import pallas as pl

N_TOKENS = 16384
D_MODEL = 1024
BLK = 1024


def _body(x_ref, wg_ref, bg_ref, o0_ref, o1_ref, o2_ref):
    xb = x_ref[...]
    z = jnp.dot(xb, wg_ref[...], preferred_element_type=jnp.float32) + bg_ref[...]
    m0 = (z[:, 0:1] > 0.0).astype(jnp.float32)
    m1 = (z[:, 1:2] > 0.0).astype(jnp.float32)
    a = xb * m0
    b = xb * m1
    o0_ref[...] = a
    o1_ref[...] = b
    o2_ref[...] = a + b


def kernel(x, Wg, bg):
    n, d = x.shape
    wg_pad = jnp.zeros((d, 128), jnp.float32).at[:, : Wg.shape[1]].set(Wg)
    bg_pad = jnp.zeros((1, 128), jnp.float32).at[0, : bg.shape[0]].set(bg)
    out_shape = jax.ShapeDtypeStruct((n, d), jnp.float32)
    grid = (n // BLK,)
    o0, o1, o2 = pl.pallas_call(
        _body,
        grid=grid,
        in_specs=[
            pl.BlockSpec((BLK, d), lambda i: (i, 0)),
            pl.BlockSpec((d, 128), lambda i: (0, 0)),
            pl.BlockSpec((1, 128), lambda i: (0, 0)),
        ],
        out_specs=[
            pl.BlockSpec((BLK, d), lambda i: (i, 0)),
            pl.BlockSpec((BLK, d), lambda i: (i, 0)),
            pl.BlockSpec((BLK, d), lambda i: (i, 0)),
        ],
        out_shape=[out_shape, out_shape, out_shape],
    )(x, wg_pad, bg_pad)
    return (o0, o1, o2)
